# baseline (speedup 1.0000x reference)
import functools

import jax
import jax.numpy as jnp
from jax import lax
from jax.experimental import pallas as pl
from jax.experimental.pallas import tpu as pltpu
from jax.experimental.pallas import tpu_sc as plsc

N = 10000
E = 320000
F = 128
H = 32
G = 64
CLS = 10

NC = 2
NS = 16
NW = NC * NS
CH = 128
K = 80
EPAD = NW * K * CH
NPAD = 10240
RP = NPAD // 4
RN = N // 4
RS = RP // NS
D = 8
NSLOT = 2 * D
KP = 5
EPOOL = NW * KP * CH


def _segsum_body(nch, full, ei_hbm, p_hbm, zeros_hbm, out_hbm, srcb, dstb,
                 rows, acc, pshr, semg, sems):
    kb, r = nch // NW, nch % NW
    P = min(D, kb)
    c = lax.axis_index("c")
    s = lax.axis_index("s")
    wid = c * NS + s
    rz = NPAD // NS
    if full:
        pltpu.sync_copy(zeros_hbm.at[pl.ds(s * rz, rz)], acc.at[pl.ds(s * rz, rz)])
    else:
        @pl.when(s == 0)
        def _():
            pltpu.sync_copy(zeros_hbm.at[pl.ds(0, rz)], acc.at[pl.ds(0, rz)])
    pltpu.sync_copy(p_hbm.at[pl.ds(s * rz, rz)], pshr.at[pl.ds(s * rz, rz)])
    if r:
        base = kb * wid + jnp.minimum(wid, r)
        cnt = kb + jnp.where(wid < r, 1, 0)
    else:
        base = kb * wid
        cnt = kb
    pltpu.sync_copy(ei_hbm.at[0, pl.ds(base, kb)], srcb.at[pl.ds(0, kb)])
    pltpu.sync_copy(ei_hbm.at[1, pl.ds(base, kb)], dstb.at[pl.ds(0, kb)])
    if r:
        @pl.when(wid < r)
        def _():
            pltpu.sync_copy(ei_hbm.at[0, pl.ds(base + kb, 1)], srcb.at[pl.ds(kb, 1)])
            pltpu.sync_copy(ei_hbm.at[1, pl.ds(base + kb, 1)], dstb.at[pl.ds(kb, 1)])
    plsc.subcore_barrier()

    hbm_side = lax.rem(s, 2) == 1

    def fire(j, slot):
        @pl.when(jnp.logical_not(hbm_side))
        def _():
            pltpu.async_copy(pshr.at[srcb.at[j]], rows.at[slot], semg)

        @pl.when(hbm_side)
        def _():
            pltpu.async_copy(p_hbm.at[srcb.at[j]], rows.at[slot], semg)

    for b in range(P):
        fire(b, b)

    def step(j, carry):
        jm = lax.rem(j, NSLOT)
        pltpu.make_async_copy(pshr.at[srcb.at[j]], rows.at[jm], semg).wait()
        pltpu.async_copy(rows.at[jm], acc.at[dstb.at[j]], sems, add=True)

        @pl.when(j >= P)
        def _():
            jd = j - P
            pltpu.make_async_copy(rows.at[lax.rem(jd, NSLOT)],
                                  acc.at[dstb.at[jd]], sems).wait()

        @pl.when(j + P < cnt)
        def _():
            fire(j + P, lax.rem(j + P, NSLOT))

        return carry

    lax.fori_loop(0, cnt, step, 0)

    def drain(j, carry):
        pltpu.make_async_copy(rows.at[lax.rem(j, NSLOT)],
                              acc.at[dstb.at[j]], sems).wait()
        return carry

    lax.fori_loop(jnp.maximum(cnt - P, 0), cnt, drain, 0)
    plsc.subcore_barrier()
    if full:
        pltpu.sync_copy(acc.at[pl.ds(s * rz, rz)], out_hbm.at[c, pl.ds(s * rz, rz)])
    else:
        @pl.when(s == 0)
        def _():
            pltpu.sync_copy(acc.at[pl.ds(0, rz)], out_hbm.at[c, pl.ds(0, rz)])


@functools.lru_cache(maxsize=4)
def _make_segsum(nch, full):
    km = nch // NW + (1 if nch % NW else 0)
    return pl.kernel(
        functools.partial(_segsum_body, nch, full),
        out_type=jax.ShapeDtypeStruct((NC, NPAD, H), jnp.float32),
        mesh=plsc.VectorSubcoreMesh(core_axis_name="c", subcore_axis_name="s"),
        scratch_types=[
            pltpu.VMEM((km, CH), jnp.int32),
            pltpu.VMEM((km, CH), jnp.int32),
            pltpu.VMEM((NSLOT, CH, H), jnp.float32),
            pltpu.VMEM_SHARED((NPAD, H), jnp.float32),
            pltpu.VMEM_SHARED((NPAD, H), jnp.float32),
            pltpu.SemaphoreType.DMA,
            pltpu.SemaphoreType.DMA,
        ],
        compiler_params=pltpu.CompilerParams(use_tc_tiling_on_sc=False),
    )


def _fold4(v, n):
    ri = lax.rem(lax.broadcasted_iota(jnp.int32, (F, F), 0), H)
    cj = lax.rem(lax.broadcasted_iota(jnp.int32, (F, F), 1), H)
    m = (ri == cj).astype(jnp.float32)
    return jnp.dot(v, m, preferred_element_type=jnp.float32) / n


def _mlp_bn(p, pa, pb, b1, w2big, b2, gam, bet):
    z = jnp.maximum(p + pa + pb + b1, 0.0)
    z = jnp.maximum(jnp.dot(z, w2big, preferred_element_type=jnp.float32) + b2, 0.0)
    zs = z[0:RN]
    mu = _fold4(jnp.sum(zs, axis=0, keepdims=True), float(N))
    zc = z - mu
    zcs = zc[0:RN]
    var = _fold4(jnp.sum(zcs * zcs, axis=0, keepdims=True), float(N))
    return zc * lax.rsqrt(var + 1e-5) * gam + bet


def _proj_body(x_ref, w1big_ref, o_ref):
    o_ref[0:RN, :] = jnp.dot(x_ref[...], w1big_ref[...],
                             preferred_element_type=jnp.float32)
    o_ref[RN:RP, :] = jnp.zeros((RP - RN, F), jnp.float32)


_proj = pl.pallas_call(_proj_body, out_shape=jax.ShapeDtypeStruct((RP, F), jnp.float32))


def _layer_body(p_ref, parts_ref, b1_ref, w2big_ref, b2_ref, g_ref, be_ref,
                w1nbig_ref, o_ref):
    h = _mlp_bn(p_ref[...], parts_ref[0], parts_ref[1], b1_ref[...],
                w2big_ref[...], b2_ref[...], g_ref[...], be_ref[...])
    o_ref[...] = jnp.dot(h, w1nbig_ref[...], preferred_element_type=jnp.float32)


_layer = pl.pallas_call(_layer_body, out_shape=jax.ShapeDtypeStruct((RP, F), jnp.float32))


def _last_body(p_ref, parts_ref, b1_ref, w2big_ref, b2_ref, g_ref, be_ref, o_ref):
    h = _mlp_bn(p_ref[...], parts_ref[0], parts_ref[1], b1_ref[...],
                w2big_ref[...], b2_ref[...], g_ref[...], be_ref[...])
    o_ref[...] = h
    o_ref[RN:RN + 1, :] = jnp.ones((1, F), jnp.float32)


_last = pl.pallas_call(_last_body, out_shape=jax.ShapeDtypeStruct((RP, F), jnp.float32))


def _unpack4(vp):
    gi = lax.broadcasted_iota(jnp.int32, (G, G // 4), 0)
    ri = lax.broadcasted_iota(jnp.int32, (G, G // 4), 1)
    out = jnp.zeros((G, H), jnp.float32)
    for a in range(4):
        ua = ((lax.rem(gi, 4) == a) & (lax.div(gi, 4) == ri)).astype(jnp.float32)
        out = out + jnp.dot(ua, vp[:, a * H:(a + 1) * H],
                            preferred_element_type=jnp.float32)
    return out


def _head_body(pp_ref, fc1w_ref, fc1b_ref, fc2w_ref, fc2b_ref, o_ref):
    spp = pp_ref[0] + pp_ref[1]
    sums = _unpack4(spp[0:G // 4])
    counts = _unpack4(spp[G // 4:2 * G // 4])[:, 0:1]
    pooled = sums / jnp.maximum(counts, 1.0)
    z = jnp.maximum(jnp.dot(pooled, fc1w_ref[...],
                            preferred_element_type=jnp.float32) + fc1b_ref[...], 0.0)
    logits = jnp.dot(z, fc2w_ref[...], preferred_element_type=jnp.float32) + fc2b_ref[...]
    m = jnp.max(logits, axis=-1, keepdims=True)
    lse = m + jnp.log(jnp.sum(jnp.exp(logits - m), axis=-1, keepdims=True))
    o_ref[...] = logits - lse


_head = pl.pallas_call(_head_body, out_shape=jax.ShapeDtypeStruct((G, CLS), jnp.float32))


def kernel(x, params, edge_index, batch):
    ei3 = edge_index.astype(jnp.int32).reshape(2, E // CH, CH)
    bat = batch.astype(jnp.int32)
    padp = EPOOL - 2 * N
    srcp = jnp.concatenate([jnp.arange(N, dtype=jnp.int32), jnp.full((N,), N, jnp.int32),
                            jnp.zeros((padp,), jnp.int32)]).reshape(NW * KP, CH)
    dstp = jnp.concatenate([bat, bat + G,
                            jnp.full((padp,), NPAD - 1, jnp.int32)]).reshape(NW * KP, CH)
    eip = jnp.stack([srcp, dstp])
    zeros = jnp.zeros((NPAD, H), jnp.float32)
    x_r = x.reshape(RN, 4 * F)
    eye4 = jnp.eye(4, dtype=jnp.float32)
    big = lambda w: jnp.kron(eye4, w)
    vec4 = lambda v: jnp.tile(v, 4).reshape(1, F)

    segsum = _make_segsum(E // CH, True)
    p = _proj(x_r, big(params["conv1_W1"]))
    for i in range(1, 6):
        parts = segsum(ei3, p.reshape(NPAD, H), zeros).reshape(NC, RP, F)
        args = (p, parts, vec4(params[f"conv{i}_b1"]), big(params[f"conv{i}_W2"]),
                vec4(params[f"conv{i}_b2"]), vec4(params[f"bn{i}_gamma"]),
                vec4(params[f"bn{i}_beta"]))
        if i < 5:
            p = _layer(*args, big(params[f"conv{i + 1}_W1"]))
        else:
            h5 = _last(*args)
    pool = _make_segsum(NW * KP, False)(eip, h5.reshape(NPAD, H), zeros)
    pp = pool.reshape(NC, RP, F)[:, 0:2 * G // 4, :]
    return _head(pp, params["fc1_W"], params["fc1_b"].reshape(1, H),
                 params["fc2_W"], params["fc2_b"].reshape(1, CLS))
